# baseline (speedup 1.0000x reference)
import functools

import jax
import jax.numpy as jnp
from jax import lax
from jax.experimental import pallas as pl
from jax.experimental.pallas import tpu as pltpu
from jax.experimental.pallas import tpu_sc as plsc

N = 10000
E = 320000
H = 128
NP = 10240
NC = 2
NS = 16
NW = NC * NS
CHUNK = 128
IG = 20
NCHUNK = 80
NG = NCHUNK // IG
EPW = NCHUNK * CHUNK
EP = NW * EPW
DUMMY = N
ROWS_PER_TILE = NP // NS
NB = NP // 512

_mesh = plsc.VectorSubcoreMesh(core_axis_name="c", subcore_axis_name="s")


@functools.partial(
    pl.kernel,
    out_type=jax.ShapeDtypeStruct((NC, NP, H), jnp.float32),
    mesh=_mesh,
    scratch_types=[
        pltpu.VMEM((IG, CHUNK), jnp.int32),
        pltpu.VMEM((IG, CHUNK), jnp.int32),
        pltpu.VMEM((IG, CHUNK), jnp.int32),
        pltpu.VMEM((IG, CHUNK), jnp.int32),
        pltpu.VMEM((CHUNK, H), jnp.float32),
        pltpu.VMEM((CHUNK, H), jnp.float32),
        pltpu.VMEM_SHARED((NP, H), jnp.float32),
        pltpu.SemaphoreType.DMA,
        pltpu.SemaphoreType.DMA,
        pltpu.SemaphoreType.DMA,
        pltpu.SemaphoreType.DMA,
        pltpu.SemaphoreType.DMA,
    ],
)
def _segsum_kernel(table_hbm, idx_hbm, dst_hbm, zeros_hbm, out_hbm,
                   idxA, dstA, idxB, dstB, rows0, rows1, acc,
                   sem0, sem1, semA, semB, semZ):
    c = lax.axis_index("c")
    s = lax.axis_index("s")
    wid = c * NS + s
    sl = pl.ds(s * ROWS_PER_TILE, ROWS_PER_TILE)
    zcp = pltpu.async_copy(zeros_hbm.at[sl], acc.at[sl], semZ)
    gbufs = ((idxA, dstA, semA), (idxB, dstB, semB))

    def load_group(gi):
        iv, dv, sg = gbufs[gi % 2]
        pltpu.async_copy(idx_hbm.at[pl.ds(gi * IG, IG), wid], iv, sg)
        pltpu.async_copy(dst_hbm.at[pl.ds(gi * IG, IG), wid], dv, sg)

    def wait_group(gi):
        iv, dv, sg = gbufs[gi % 2]
        pltpu.make_async_copy(idx_hbm.at[pl.ds(gi * IG, IG), wid], iv, sg).wait()
        pltpu.make_async_copy(dst_hbm.at[pl.ds(gi * IG, IG), wid], dv, sg).wait()
        return iv, dv

    load_group(0)
    zcp.wait()
    plsc.subcore_barrier()

    bufs = ((rows0, sem0), (rows1, sem1))
    for gi in range(NG):
        idx_v, dst_v = wait_group(gi)
        if gi + 1 < NG:
            load_group(gi + 1)
        pltpu.async_copy(table_hbm.at[idx_v.at[0]], rows0, sem0)
        for k in range(IG):
            buf, sem = bufs[k % 2]
            if k + 1 < IG:
                obuf, osem = bufs[(k + 1) % 2]
                pltpu.async_copy(table_hbm.at[idx_v.at[k + 1]], obuf, osem)
            pltpu.make_async_copy(table_hbm.at[idx_v.at[k]], buf, sem).wait()
            pltpu.sync_copy(buf, acc.at[dst_v.at[k]], add=True)

    plsc.subcore_barrier()
    pltpu.sync_copy(acc.at[sl], out_hbm.at[c].at[sl])


def _atom_body(hn_ref, emb_ref, out_ref):
    iota = lax.broadcasted_iota(jnp.int32, (512, 64), 1)
    oh = jnp.concatenate(
        [(hn_ref[:, f:f + 1] == iota).astype(jnp.float32) for f in range(9)],
        axis=1)
    out_ref[...] = jnp.dot(oh, emb_ref[...], preferred_element_type=jnp.float32)


def _atom_encode(h_node_p, atom_flat):
    return pl.pallas_call(
        _atom_body,
        grid=(NB,),
        in_specs=[
            pl.BlockSpec((512, 16), lambda i: (i, 0)),
            pl.BlockSpec((9 * 64, H), lambda i: (0, 0)),
        ],
        out_specs=pl.BlockSpec((512, H), lambda i: (i, 0)),
        out_shape=jax.ShapeDtypeStruct((NP, H), jnp.float32),
    )(h_node_p, atom_flat)


def _dense_body(relu, h_ref, p_ref, c_ref, bond_ref,
                w_ref, b_ref, g_ref, bb_ref, pw_ref, pb_ref,
                out_ref, pool_ref, pred_ref):
    i = pl.program_id(0)
    h = h_ref[...]
    cnt = c_ref[0] + c_ref[1]
    deg = jnp.maximum(jnp.sum(cnt[:, 0:8], axis=1, keepdims=True), 1.0)
    neigh = (p_ref[0] + p_ref[1]
             + jnp.dot(cnt, bond_ref[...], preferred_element_type=jnp.float32))
    rst = h + neigh / deg
    y = jnp.dot(rst, w_ref[...], preferred_element_type=jnp.float32) + b_ref[...]
    mu = jnp.mean(y, axis=-1, keepdims=True)
    d = y - mu
    var = jnp.mean(d * d, axis=-1, keepdims=True)
    y = d * lax.rsqrt(var + 1e-5) * g_ref[...] + bb_ref[...]
    if relu:
        y = jnp.maximum(y, 0.0)
    row = i * 512 + lax.broadcasted_iota(jnp.int32, (512, 1), 0)
    out = (y + h) * (row < N).astype(jnp.float32)
    out_ref[...] = out

    @pl.when(i == 0)
    def _():
        pool_ref[...] = jnp.zeros((1, H), jnp.float32)

    pool_ref[...] += jnp.sum(out, axis=0, keepdims=True)

    @pl.when(i == NB - 1)
    def _():
        pooled = pool_ref[...] * (1.0 / N)
        pred_ref[...] = (jnp.dot(pooled, pw_ref[...],
                                 preferred_element_type=jnp.float32) + pb_ref[...])


def _dense_stage(relu, h, part, cnt, bond, w, b, g, bb, pw, pb):
    full = lambda *shape: pl.BlockSpec(shape, lambda i: tuple(0 for _ in shape))
    return pl.pallas_call(
        functools.partial(_dense_body, relu),
        grid=(NB,),
        in_specs=[
            pl.BlockSpec((512, H), lambda i: (i, 0)),
            pl.BlockSpec((2, 512, H), lambda i: (0, i, 0)),
            pl.BlockSpec((2, 512, H), lambda i: (0, i, 0)),
            full(H, H),
            full(H, H),
            full(1, H),
            full(1, H),
            full(1, H),
            full(H, H),
            full(1, H),
        ],
        out_specs=[
            pl.BlockSpec((512, H), lambda i: (i, 0)),
            pl.BlockSpec((1, H), lambda i: (0, 0)),
            pl.BlockSpec((1, H), lambda i: (0, 0)),
        ],
        out_shape=[
            jax.ShapeDtypeStruct((NP, H), jnp.float32),
            jax.ShapeDtypeStruct((1, H), jnp.float32),
            jax.ShapeDtypeStruct((1, H), jnp.float32),
        ],
    )(h, part, cnt, bond, w, b, g, bb, pw, pb)


def kernel(edge_index, h_node, h_edge, atom_emb, bond_emb, lin_W, lin_b,
           ln_g, ln_b, pred_W, pred_b):
    L = lin_W.shape[0]
    pad_e = EP - E
    pad_ar = jnp.arange(pad_e, dtype=jnp.int32)
    deal = lambda a: a.reshape(NCHUNK, NW, CHUNK)
    src3 = deal(jnp.concatenate([edge_index[0], N + pad_ar % (NP - N)]))
    dst3 = deal(jnp.concatenate([edge_index[1], N + pad_ar % (NP - N)]))
    code = (h_edge[:, 0] + 8 * h_edge[:, 1] + 64 * h_edge[:, 2]).astype(jnp.int32)
    rep = 512 * (jnp.arange(EP, dtype=jnp.int32) % 8)
    code3 = deal(jnp.concatenate([code, pad_ar % 512]) + rep)
    k = jnp.arange(512, dtype=jnp.int32)
    comb = jnp.tile(
        jax.nn.one_hot(k % 8, H, dtype=jnp.float32)
        + jax.nn.one_hot(8 + (k // 8) % 8, H, dtype=jnp.float32)
        + jax.nn.one_hot(16 + k // 64, H, dtype=jnp.float32), (8, 1))
    hn_p = jnp.full((NP, 16), -1, jnp.int32).at[:N, :9].set(h_node)
    zh = jnp.zeros((NP, H), jnp.float32)
    bond_flat = jnp.zeros((L, H, H), jnp.float32).at[:, :24].set(
        bond_emb.reshape(L, 24, H))

    cnt = _segsum_kernel(comb, code3, dst3, zh)
    h = _atom_encode(hn_p, atom_emb.reshape(9 * 64, H))
    pred = None
    for i in range(L):
        part = _segsum_kernel(h, src3, dst3, zh)
        h, _, pred = _dense_stage(
            i != L - 1, h, part, cnt, bond_flat[i],
            lin_W[i], lin_b[i].reshape(1, H), ln_g[i].reshape(1, H),
            ln_b[i].reshape(1, H), pred_W, pred_b.reshape(1, H))
    return pred


# speedup vs baseline: 1.0151x; 1.0151x over previous
import functools

import jax
import jax.numpy as jnp
from jax import lax
from jax.experimental import pallas as pl
from jax.experimental.pallas import tpu as pltpu
from jax.experimental.pallas import tpu_sc as plsc

N = 10000
E = 320000
H = 128
NP = 10240
NC = 2
NS = 16
NW = NC * NS
CHUNK = 128
IG = 40
NCHUNK = 80
NG = NCHUNK // IG
EPW = NCHUNK * CHUNK
EP = NW * EPW
DUMMY = N
ROWS_PER_TILE = NP // NS
NB = NP // 512

_mesh = plsc.VectorSubcoreMesh(core_axis_name="c", subcore_axis_name="s")


@functools.partial(
    pl.kernel,
    out_type=jax.ShapeDtypeStruct((NC, NP, H), jnp.float32),
    mesh=_mesh,
    scratch_types=[
        pltpu.VMEM((IG, CHUNK), jnp.int32),
        pltpu.VMEM((IG, CHUNK), jnp.int32),
        pltpu.VMEM((CHUNK, H), jnp.float32),
        pltpu.VMEM((CHUNK, H), jnp.float32),
        pltpu.VMEM_SHARED((NP, H), jnp.float32),
        pltpu.SemaphoreType.DMA,
        pltpu.SemaphoreType.DMA,
    ],
)
def _segsum_kernel(table_hbm, idx_hbm, dst_hbm, zeros_hbm, out_hbm,
                   idx_v, dst_v, rows0, rows1, acc, sem0, sem1):
    c = lax.axis_index("c")
    s = lax.axis_index("s")
    wid = c * NS + s
    sl = pl.ds(s * ROWS_PER_TILE, ROWS_PER_TILE)
    pltpu.sync_copy(zeros_hbm.at[sl], acc.at[sl])
    plsc.subcore_barrier()

    bufs = ((rows0, sem0), (rows1, sem1))

    def group(gi, carry):
        gs = pl.ds(gi * IG, IG)
        pltpu.sync_copy(idx_hbm.at[gs, wid], idx_v)
        pltpu.sync_copy(dst_hbm.at[gs, wid], dst_v)
        pltpu.async_copy(table_hbm.at[idx_v.at[0]], rows0, sem0)
        for k in range(IG):
            buf, sem = bufs[k % 2]
            if k + 1 < IG:
                obuf, osem = bufs[(k + 1) % 2]
                pltpu.async_copy(table_hbm.at[idx_v.at[k + 1]], obuf, osem)
            pltpu.make_async_copy(table_hbm.at[idx_v.at[k]], buf, sem).wait()
            pltpu.sync_copy(buf, acc.at[dst_v.at[k]], add=True)
        return carry

    lax.fori_loop(0, NG, group, 0)
    plsc.subcore_barrier()
    pltpu.sync_copy(acc.at[sl], out_hbm.at[c].at[sl])


def _atom_body(hn_ref, emb_ref, out_ref):
    iota = lax.broadcasted_iota(jnp.int32, (512, 64), 1)
    oh = jnp.concatenate(
        [(hn_ref[:, f:f + 1] == iota).astype(jnp.float32) for f in range(9)],
        axis=1)
    out_ref[...] = jnp.dot(oh, emb_ref[...], preferred_element_type=jnp.float32)


def _atom_encode(h_node_p, atom_flat):
    return pl.pallas_call(
        _atom_body,
        grid=(NB,),
        in_specs=[
            pl.BlockSpec((512, 16), lambda i: (i, 0)),
            pl.BlockSpec((9 * 64, H), lambda i: (0, 0)),
        ],
        out_specs=pl.BlockSpec((512, H), lambda i: (i, 0)),
        out_shape=jax.ShapeDtypeStruct((NP, H), jnp.float32),
    )(h_node_p, atom_flat)


def _dense_body(relu, h_ref, p_ref, c_ref, bond_ref,
                w_ref, b_ref, g_ref, bb_ref, pw_ref, pb_ref,
                out_ref, pool_ref, pred_ref):
    i = pl.program_id(0)
    h = h_ref[...]
    cnt = c_ref[0] + c_ref[1]
    deg = jnp.maximum(jnp.sum(cnt[:, 0:8], axis=1, keepdims=True), 1.0)
    neigh = (p_ref[0] + p_ref[1]
             + jnp.dot(cnt, bond_ref[...], preferred_element_type=jnp.float32))
    rst = h + neigh / deg
    y = jnp.dot(rst, w_ref[...], preferred_element_type=jnp.float32) + b_ref[...]
    mu = jnp.mean(y, axis=-1, keepdims=True)
    d = y - mu
    var = jnp.mean(d * d, axis=-1, keepdims=True)
    y = d * lax.rsqrt(var + 1e-5) * g_ref[...] + bb_ref[...]
    if relu:
        y = jnp.maximum(y, 0.0)
    row = i * 512 + lax.broadcasted_iota(jnp.int32, (512, 1), 0)
    out = (y + h) * (row < N).astype(jnp.float32)
    out_ref[...] = out

    @pl.when(i == 0)
    def _():
        pool_ref[...] = jnp.zeros((1, H), jnp.float32)

    pool_ref[...] += jnp.sum(out, axis=0, keepdims=True)

    @pl.when(i == NB - 1)
    def _():
        pooled = pool_ref[...] * (1.0 / N)
        pred_ref[...] = (jnp.dot(pooled, pw_ref[...],
                                 preferred_element_type=jnp.float32) + pb_ref[...])


def _dense_stage(relu, h, part, cnt, bond, w, b, g, bb, pw, pb):
    full = lambda *shape: pl.BlockSpec(shape, lambda i: tuple(0 for _ in shape))
    return pl.pallas_call(
        functools.partial(_dense_body, relu),
        grid=(NB,),
        in_specs=[
            pl.BlockSpec((512, H), lambda i: (i, 0)),
            pl.BlockSpec((2, 512, H), lambda i: (0, i, 0)),
            pl.BlockSpec((2, 512, H), lambda i: (0, i, 0)),
            full(H, H),
            full(H, H),
            full(1, H),
            full(1, H),
            full(1, H),
            full(H, H),
            full(1, H),
        ],
        out_specs=[
            pl.BlockSpec((512, H), lambda i: (i, 0)),
            pl.BlockSpec((1, H), lambda i: (0, 0)),
            pl.BlockSpec((1, H), lambda i: (0, 0)),
        ],
        out_shape=[
            jax.ShapeDtypeStruct((NP, H), jnp.float32),
            jax.ShapeDtypeStruct((1, H), jnp.float32),
            jax.ShapeDtypeStruct((1, H), jnp.float32),
        ],
    )(h, part, cnt, bond, w, b, g, bb, pw, pb)


def kernel(edge_index, h_node, h_edge, atom_emb, bond_emb, lin_W, lin_b,
           ln_g, ln_b, pred_W, pred_b):
    L = lin_W.shape[0]
    pad_e = EP - E
    pad_ar = jnp.arange(pad_e, dtype=jnp.int32)
    deal = lambda a: a.reshape(NCHUNK, NW, CHUNK)
    src3 = deal(jnp.concatenate([edge_index[0], N + pad_ar % (NP - N)]))
    dst3 = deal(jnp.concatenate([edge_index[1], N + pad_ar % (NP - N)]))
    code = (h_edge[:, 0] + 8 * h_edge[:, 1] + 64 * h_edge[:, 2]).astype(jnp.int32)
    rep = 512 * (jnp.arange(EP, dtype=jnp.int32) % 8)
    code3 = deal(jnp.concatenate([code, pad_ar % 512]) + rep)
    k = jnp.arange(512, dtype=jnp.int32)
    comb = jnp.tile(
        jax.nn.one_hot(k % 8, H, dtype=jnp.float32)
        + jax.nn.one_hot(8 + (k // 8) % 8, H, dtype=jnp.float32)
        + jax.nn.one_hot(16 + k // 64, H, dtype=jnp.float32), (8, 1))
    hn_p = jnp.full((NP, 16), -1, jnp.int32).at[:N, :9].set(h_node)
    zh = jnp.zeros((NP, H), jnp.float32)
    bond_flat = jnp.zeros((L, H, H), jnp.float32).at[:, :24].set(
        bond_emb.reshape(L, 24, H))

    cnt = _segsum_kernel(comb, code3, dst3, zh)
    h = _atom_encode(hn_p, atom_emb.reshape(9 * 64, H))
    pred = None
    for i in range(L):
        part = _segsum_kernel(h, src3, dst3, zh)
        h, _, pred = _dense_stage(
            i != L - 1, h, part, cnt, bond_flat[i],
            lin_W[i], lin_b[i].reshape(1, H), ln_g[i].reshape(1, H),
            ln_b[i].reshape(1, H), pred_W, pred_b.reshape(1, H))
    return pred


# speedup vs baseline: 1.0289x; 1.0135x over previous
import functools

import jax
import jax.numpy as jnp
from jax import lax
from jax.experimental import pallas as pl
from jax.experimental.pallas import tpu as pltpu
from jax.experimental.pallas import tpu_sc as plsc

N = 10000
E = 320000
H = 128
NP = 10240
NC = 2
NS = 16
NW = NC * NS
CHUNK = 128
IG = 40
NCHUNK = 80
NG = NCHUNK // IG
EPW = NCHUNK * CHUNK
EP = NW * EPW
DUMMY = N
ROWS_PER_TILE = NP // NS
NB = NP // 512

_mesh = plsc.VectorSubcoreMesh(core_axis_name="c", subcore_axis_name="s")


def _make_segsum(pair_src, ig):
    ng = NCHUNK // ig
    scratch = [
        pltpu.VMEM((ig, CHUNK), jnp.int32),
        pltpu.VMEM((ig, CHUNK), jnp.int32),
        pltpu.VMEM((CHUNK, H), jnp.float32),
        pltpu.VMEM((CHUNK, H), jnp.float32),
        pltpu.VMEM_SHARED((NP, H), jnp.float32),
        pltpu.SemaphoreType.DMA,
        pltpu.SemaphoreType.DMA,
    ]

    @functools.partial(
        pl.kernel,
        out_type=jax.ShapeDtypeStruct((NC, NP, H), jnp.float32),
        mesh=_mesh,
        scratch_types=scratch,
    )
    def segsum(table_hbm, src_hbm, ei_hbm, zeros_hbm, out_hbm,
               idx_v, dst_v, rows0, rows1, acc, sem0, sem1):
        c = lax.axis_index("c")
        s = lax.axis_index("s")
        wid = c * NS + s
        sl = pl.ds(s * ROWS_PER_TILE, ROWS_PER_TILE)
        pltpu.sync_copy(zeros_hbm.at[sl], acc.at[sl])
        plsc.subcore_barrier()

        bufs = ((rows0, sem0), (rows1, sem1))

        def group(gi, carry):
            gs = pl.ds(gi * ig, ig)
            if pair_src:
                pltpu.sync_copy(src_hbm.at[0, gs, wid], idx_v)
            else:
                pltpu.sync_copy(src_hbm.at[gs, wid], idx_v)
            pltpu.sync_copy(ei_hbm.at[1, gs, wid], dst_v)
            pltpu.async_copy(table_hbm.at[idx_v.at[0]], rows0, sem0)
            for k in range(ig):
                buf, sem = bufs[k % 2]
                if k + 1 < ig:
                    obuf, osem = bufs[(k + 1) % 2]
                    pltpu.async_copy(table_hbm.at[idx_v.at[k + 1]], obuf, osem)
                pltpu.make_async_copy(table_hbm.at[idx_v.at[k]], buf, sem).wait()
                pltpu.sync_copy(buf, acc.at[dst_v.at[k]], add=True)
            return carry

        lax.fori_loop(0, ng, group, 0)
        plsc.subcore_barrier()
        pltpu.sync_copy(acc.at[sl], out_hbm.at[c].at[sl])

    return segsum


_segsum_kernel = _make_segsum(True, IG)
_counts_segsum = _make_segsum(False, IG)


def _atom_body(hn_ref, emb_ref, out_ref):
    iota = lax.broadcasted_iota(jnp.int32, (512, 64), 1)
    oh = jnp.concatenate(
        [(hn_ref[:, f:f + 1] == iota).astype(jnp.float32) for f in range(9)],
        axis=1)
    out_ref[...] = jnp.dot(oh, emb_ref[...], preferred_element_type=jnp.float32)


def _atom_encode(h_node_p, atom_flat):
    return pl.pallas_call(
        _atom_body,
        grid=(NB,),
        in_specs=[
            pl.BlockSpec((512, 16), lambda i: (i, 0)),
            pl.BlockSpec((9 * 64, H), lambda i: (0, 0)),
        ],
        out_specs=pl.BlockSpec((512, H), lambda i: (i, 0)),
        out_shape=jax.ShapeDtypeStruct((NP, H), jnp.float32),
    )(h_node_p, atom_flat)


def _bondprep_body(c_ref, bond_ref, rdeg_ref, cb_ref):
    cnt = c_ref[0] + c_ref[1]
    deg = jnp.maximum(jnp.sum(cnt[:, 0:8], axis=1, keepdims=True), 1.0)
    rdeg = 1.0 / deg
    rdeg_ref[...] = jnp.broadcast_to(rdeg, (512, H))
    for i in range(2):
        cb_ref[i] = jnp.dot(cnt, bond_ref[i],
                            preferred_element_type=jnp.float32) * rdeg


def _bondprep(cnt, bond_flat):
    return pl.pallas_call(
        _bondprep_body,
        grid=(NB,),
        in_specs=[
            pl.BlockSpec((2, 512, H), lambda i: (0, i, 0)),
            pl.BlockSpec((2, H, H), lambda i: (0, 0, 0)),
        ],
        out_specs=[
            pl.BlockSpec((512, H), lambda i: (i, 0)),
            pl.BlockSpec((2, 512, H), lambda i: (0, i, 0)),
        ],
        out_shape=[
            jax.ShapeDtypeStruct((NP, H), jnp.float32),
            jax.ShapeDtypeStruct((2, NP, H), jnp.float32),
        ],
    )(cnt, bond_flat)


def _dense_body(relu, h_ref, p_ref, rdeg_ref, cb_ref,
                w_ref, b_ref, g_ref, bb_ref, pw_ref, pb_ref,
                out_ref, pool_ref, pred_ref):
    i = pl.program_id(0)
    h = h_ref[...]
    rst = h + (p_ref[0] + p_ref[1]) * rdeg_ref[...] + cb_ref[0]
    y = jnp.dot(rst, w_ref[...], preferred_element_type=jnp.float32) + b_ref[...]
    mu = jnp.mean(y, axis=-1, keepdims=True)
    d = y - mu
    var = jnp.mean(d * d, axis=-1, keepdims=True)
    y = d * lax.rsqrt(var + 1e-5) * g_ref[...] + bb_ref[...]
    if relu:
        y = jnp.maximum(y, 0.0)
    row = i * 512 + lax.broadcasted_iota(jnp.int32, (512, 1), 0)
    out = (y + h) * (row < N).astype(jnp.float32)
    out_ref[...] = out

    @pl.when(i == 0)
    def _():
        pool_ref[...] = jnp.zeros((1, H), jnp.float32)

    pool_ref[...] += jnp.sum(out, axis=0, keepdims=True)

    @pl.when(i == NB - 1)
    def _():
        pooled = pool_ref[...] * (1.0 / N)
        pred_ref[...] = (jnp.dot(pooled, pw_ref[...],
                                 preferred_element_type=jnp.float32) + pb_ref[...])


def _dense_stage(relu, li, h, part, rdeg, cb, w, b, g, bb, pw, pb):
    full = lambda *shape: pl.BlockSpec(shape, lambda i: tuple(0 for _ in shape))
    return pl.pallas_call(
        functools.partial(_dense_body, relu),
        grid=(NB,),
        in_specs=[
            pl.BlockSpec((512, H), lambda i: (i, 0)),
            pl.BlockSpec((2, 512, H), lambda i: (0, i, 0)),
            pl.BlockSpec((512, H), lambda i: (i, 0)),
            pl.BlockSpec((1, 512, H), lambda i: (li, i, 0)),
            full(H, H),
            full(1, H),
            full(1, H),
            full(1, H),
            full(H, H),
            full(1, H),
        ],
        out_specs=[
            pl.BlockSpec((512, H), lambda i: (i, 0)),
            pl.BlockSpec((1, H), lambda i: (0, 0)),
            pl.BlockSpec((1, H), lambda i: (0, 0)),
        ],
        out_shape=[
            jax.ShapeDtypeStruct((NP, H), jnp.float32),
            jax.ShapeDtypeStruct((1, H), jnp.float32),
            jax.ShapeDtypeStruct((1, H), jnp.float32),
        ],
    )(h, part, rdeg, cb, w, b, g, bb, pw, pb)


def kernel(edge_index, h_node, h_edge, atom_emb, bond_emb, lin_W, lin_b,
           ln_g, ln_b, pred_W, pred_b):
    L = lin_W.shape[0]
    pad_e = EP - E
    pad_ar = jnp.arange(pad_e, dtype=jnp.int32)
    pad2 = jnp.broadcast_to(N + pad_ar % (NP - N), (2, pad_e))
    ei3 = jnp.concatenate([edge_index, pad2], axis=1).reshape(
        2, NCHUNK, NW, CHUNK)
    code = (h_edge[:, 0] + 8 * h_edge[:, 1] + 64 * h_edge[:, 2]).astype(jnp.int32)
    rep = 512 * (jnp.arange(EP, dtype=jnp.int32) % 8)
    code3 = (jnp.concatenate([code, pad_ar % 512]) + rep).reshape(
        NCHUNK, NW, CHUNK)
    k = jnp.arange(512, dtype=jnp.int32)
    comb = jnp.tile(
        jax.nn.one_hot(k % 8, H, dtype=jnp.float32)
        + jax.nn.one_hot(8 + (k // 8) % 8, H, dtype=jnp.float32)
        + jax.nn.one_hot(16 + k // 64, H, dtype=jnp.float32), (8, 1))
    hn_p = jnp.full((NP, 16), -1, jnp.int32).at[:N, :9].set(h_node)
    zh = jnp.zeros((NP, H), jnp.float32)
    bond_flat = jnp.zeros((L, H, H), jnp.float32).at[:, :24].set(
        bond_emb.reshape(L, 24, H))

    cnt = _counts_segsum(comb, code3, ei3, zh)
    h = _atom_encode(hn_p, atom_emb.reshape(9 * 64, H))
    rdeg, cb = _bondprep(cnt, bond_flat)
    pred = None
    for i in range(L):
        part = _segsum_kernel(h, ei3, ei3, zh)
        h, _, pred = _dense_stage(
            i != L - 1, i, h, part, rdeg, cb,
            lin_W[i], lin_b[i].reshape(1, H), ln_g[i].reshape(1, H),
            ln_b[i].reshape(1, H), pred_W, pred_b.reshape(1, H))
    return pred
